# baseline (speedup 1.0000x reference)
import functools

import jax
import jax.numpy as jnp
from jax import lax
from jax.experimental import pallas as pl
from jax.experimental.pallas import tpu as pltpu
from jax.experimental.pallas import tpu_sc as plsc

_N_NODES = 10000
_N_HYPER = 2000
_N_TOTAL = _N_NODES + _N_HYPER
_E = 384000
_D = 128
_C = 64
_L = 3

_NC = 2
_NS = 16
_NW = _NC * _NS
_EPW = _E // _NW
_CH = 96
_NCHUNK = _EPW // _CH
_NP = _NCHUNK // 2
_TAIL = _NCHUNK - 2 * _NP
_RPS = 752
_RPS_LAST = _N_TOTAL - 15 * _RPS

_GDN = lax.GatherDimensionNumbers(
    offset_dims=(), collapsed_slice_dims=(0,), start_index_map=(0,))


def _bcast_lane(vec16, j):
    return lax.gather(vec16, jnp.full((16, 1), j, jnp.int32), _GDN,
                      slice_sizes=(1,),
                      mode=lax.GatherScatterMode.PROMISE_IN_BOUNDS)


def _sc_layer(table, edata, ddata, zeros):
    mesh = plsc.VectorSubcoreMesh(core_axis_name="c", subcore_axis_name="s")
    ngrp = _CH // 16

    @functools.partial(
        pl.kernel,
        mesh=mesh,
        out_type=jax.ShapeDtypeStruct((_NC, _N_TOTAL, _D), jnp.float32),
        scratch_types=[
            pltpu.VMEM((2, _CH), jnp.int32),
            pltpu.VMEM((2, _CH), jnp.int32),
            pltpu.VMEM((1, _CH), jnp.int32),
            pltpu.VMEM((1, _CH), jnp.int32),
            pltpu.VMEM((_CH, _D), jnp.float32),
            pltpu.VMEM((_CH, _D), jnp.float32),
            pltpu.VMEM_SHARED((_N_TOTAL, _D), jnp.float32),
            pltpu.SemaphoreType.DMA,
            pltpu.SemaphoreType.DMA,
            pltpu.SemaphoreType.DMA,
            pltpu.SemaphoreType.DMA,
            pltpu.SemaphoreType.DMA,
            pltpu.SemaphoreType.DMA,
            pltpu.SemaphoreType.DMA,
            pltpu.SemaphoreType.DMA,
        ],
    )
    def k(table_hbm, ed_hbm, dd_hbm, zeros_hbm, out_hbm,
          ed0, ed1, dd0, dd1, rows0, rows1, acc,
          gs0, gs1, ss0, ss1, xs0, xs1, ys0, ys1):
        c = lax.axis_index("c")
        s = lax.axis_index("s")
        wid = s * _NC + c
        row0 = pl.multiple_of(s * _RPS, 8)

        @pl.when(s < _NS - 1)
        def _():
            pltpu.sync_copy(zeros_hbm.at[pl.ds(row0, _RPS)],
                            acc.at[pl.ds(row0, _RPS)])

        @pl.when(s == _NS - 1)
        def _():
            pltpu.sync_copy(zeros_hbm.at[pl.ds(15 * _RPS, _RPS_LAST)],
                            acc.at[pl.ds(15 * _RPS, _RPS_LAST)])

        plsc.subcore_barrier()

        def scale(rows, ed):
            for g in range(ngrp):
                w16 = ed[1, pl.ds(g * 16, 16)].astype(jnp.float32) * (1.0 / (1 << 24))
                for j in range(16):
                    e = g * 16 + j
                    bw = _bcast_lane(w16, j)
                    for d16 in range(_D // 16):
                        sl = pl.ds(d16 * 16, 16)
                        rows[e, sl] = rows[e, sl] * bw

        def scatter(rows, dd, sem):
            pltpu.async_copy(rows, acc.at[dd.at[0]], sem, add=True)

        def drain_scatter(rows, dd, sem):
            pltpu.make_async_copy(rows, acc.at[dd.at[0]], sem).wait()

        pltpu.sync_copy(ed_hbm.at[wid, 0], ed0)
        pltpu.sync_copy(ed_hbm.at[wid, 1], ed1)
        pltpu.sync_copy(dd_hbm.at[wid, 0], dd0)
        pltpu.sync_copy(dd_hbm.at[wid, 1], dd1)
        pltpu.async_copy(table_hbm.at[ed0.at[0]], rows0, gs0)

        def pair(p, carry):
            i0 = 2 * p
            i1 = i0 + 1

            @pl.when(p > 0)
            def _():
                drain_scatter(rows1, dd1, ss1)
                pltpu.async_copy(dd_hbm.at[wid, i1], dd1, ys1)
                pltpu.make_async_copy(ed_hbm.at[wid, i1], ed1, xs1).wait()

            pltpu.async_copy(table_hbm.at[ed1.at[0]], rows1, gs1)

            pltpu.make_async_copy(table_hbm.at[ed0.at[0]], rows0, gs0).wait()
            scale(rows0, ed0)

            @pl.when(i0 + 2 < _NCHUNK)
            def _():
                pltpu.async_copy(ed_hbm.at[wid, i0 + 2], ed0, xs0)

            @pl.when(p > 0)
            def _():
                pltpu.make_async_copy(dd_hbm.at[wid, i0], dd0, ys0).wait()

            scatter(rows0, dd0, ss0)

            pltpu.make_async_copy(table_hbm.at[ed1.at[0]], rows1, gs1).wait()
            scale(rows1, ed1)
            drain_scatter(rows0, dd0, ss0)

            @pl.when(i0 + 2 < _NCHUNK)
            def _():
                pltpu.async_copy(dd_hbm.at[wid, i0 + 2], dd0, ys0)
                pltpu.make_async_copy(ed_hbm.at[wid, i0 + 2], ed0, xs0).wait()
                pltpu.async_copy(table_hbm.at[ed0.at[0]], rows0, gs0)

            @pl.when(p > 0)
            def _():
                pltpu.make_async_copy(dd_hbm.at[wid, i1], dd1, ys1).wait()

            scatter(rows1, dd1, ss1)

            @pl.when(i1 + 2 < _NCHUNK)
            def _():
                pltpu.async_copy(ed_hbm.at[wid, i1 + 2], ed1, xs1)

            return carry

        lax.fori_loop(0, _NP, pair, 0)
        if _TAIL:
            pltpu.make_async_copy(table_hbm.at[ed0.at[0]], rows0, gs0).wait()
            scale(rows0, ed0)
            pltpu.make_async_copy(dd_hbm.at[wid, _NCHUNK - 1], dd0, ys0).wait()
            scatter(rows0, dd0, ss0)
            drain_scatter(rows0, dd0, ss0)
        drain_scatter(rows1, dd1, ss1)
        plsc.subcore_barrier()

        @pl.when(s < _NS - 1)
        def _():
            pltpu.sync_copy(acc.at[pl.ds(row0, _RPS)],
                            out_hbm.at[c, pl.ds(row0, _RPS)])

        @pl.when(s == _NS - 1)
        def _():
            pltpu.sync_copy(acc.at[pl.ds(15 * _RPS, _RPS_LAST)],
                            out_hbm.at[c, pl.ds(15 * _RPS, _RPS_LAST)])

    return k(table, edata, ddata, zeros)


def _combine(partials):
    br = 1200

    def body(p_ref, table_ref):
        table_ref[...] = p_ref[0] + p_ref[1]

    return pl.pallas_call(
        body,
        grid=(_N_TOTAL // br,),
        in_specs=[pl.BlockSpec((2, br, _D), lambda i: (0, i, 0))],
        out_specs=pl.BlockSpec((br, _D), lambda i: (i, 0)),
        out_shape=jax.ShapeDtypeStruct((_N_TOTAL, _D), jnp.float32),
    )(partials)


def _head(t0, t1, t2, p3, w, b2):
    br = 2000

    def body(t0_ref, t1_ref, t2_ref, p3_ref, w_ref, b_ref, o_ref):
        x = (t0_ref[...] + t1_ref[...] + t2_ref[...]
             + p3_ref[0] + p3_ref[1]) * (1.0 / (_L + 1))
        z = lax.dot_general(x, w_ref[...], (((1,), (1,)), ((), ())),
                            preferred_element_type=jnp.float32)
        z = z + b_ref[...]
        m = jnp.max(z, axis=1, keepdims=True)
        lse = jnp.log(jnp.sum(jnp.exp(z - m), axis=1, keepdims=True)) + m
        o_ref[...] = z - lse

    row_spec = pl.BlockSpec((br, _D), lambda i: (i, 0))
    return pl.pallas_call(
        body,
        grid=(_N_NODES // br,),
        in_specs=[
            row_spec,
            row_spec,
            row_spec,
            pl.BlockSpec((2, br, _D), lambda i: (0, i, 0)),
            pl.BlockSpec((_C, _D), lambda i: (0, 0)),
            pl.BlockSpec((1, _C), lambda i: (0, 0)),
        ],
        out_specs=pl.BlockSpec((br, _C), lambda i: (i, 0)),
        out_shape=jax.ShapeDtypeStruct((_N_NODES, _C), jnp.float32),
    )(t0, t1, t2, p3, w, b2)


def kernel(node_emb, hyperedge_emb, W, b, edge_weight, edge_index):
    table = jnp.concatenate([node_emb, hyperedge_emb], axis=0)
    dst = edge_index[0].astype(jnp.int32).reshape(_NW, _NCHUNK, _CH)
    src = edge_index[1].astype(jnp.int32).reshape(_NW, _NCHUNK, _CH)
    wbits = jnp.round(edge_weight.astype(jnp.float32) * (1 << 24)).astype(
        jnp.int32).reshape(_NW, _NCHUNK, _CH)
    edata = jnp.stack([src, wbits], axis=2)
    ddata = dst.reshape(_NW, _NCHUNK, 1, _CH)
    zeros = jnp.zeros((_N_TOTAL, _D), jnp.float32)

    tables = [table]
    for _ in range(_L - 1):
        partials = _sc_layer(tables[-1], edata, ddata, zeros)
        tables.append(_combine(partials))
    p3 = _sc_layer(tables[-1], edata, ddata, zeros)

    n = _N_NODES
    return _head(tables[0][:n], tables[1][:n], tables[2][:n], p3[:, :n],
                 W, jnp.reshape(b, (1, _C)))


# speedup vs baseline: 1.2235x; 1.2235x over previous
import functools

import jax
import jax.numpy as jnp
from jax import lax
from jax.experimental import pallas as pl
from jax.experimental.pallas import tpu as pltpu
from jax.experimental.pallas import tpu_sc as plsc

_N_NODES = 10000
_N_HYPER = 2000
_N_TOTAL = _N_NODES + _N_HYPER
_E = 384000
_D = 128
_C = 64
_L = 3

_NC = 2
_NS = 16
_NW = _NC * _NS
_EPW = _E // _NW
_CH = 80
_NCHUNK = _EPW // _CH
_NP = _NCHUNK // 2
_RPS = 752
_RPS_LAST = _N_TOTAL - 15 * _RPS

_GDN = lax.GatherDimensionNumbers(
    offset_dims=(), collapsed_slice_dims=(0,), start_index_map=(0,))


_CH2 = _CH // 2


def _bcast_lane(vec16, j):
    return lax.gather(vec16, jnp.full((16, 1), j, jnp.int32), _GDN,
                      slice_sizes=(1,),
                      mode=lax.GatherScatterMode.PROMISE_IN_BOUNDS)


def _sc_layer(table, edata, ddata, zeros):
    mesh = plsc.VectorSubcoreMesh(core_axis_name="c", subcore_axis_name="s")
    ngrp = _CH // 16

    @functools.partial(
        pl.kernel,
        mesh=mesh,
        out_type=jax.ShapeDtypeStruct((_NC, _N_TOTAL, _D), jnp.float32),
        scratch_types=[
            pltpu.VMEM((2, _CH), jnp.int32),
            pltpu.VMEM((2, _CH), jnp.int32),
            pltpu.VMEM((1, _CH), jnp.int32),
            pltpu.VMEM((1, _CH), jnp.int32),
            pltpu.VMEM((_CH, _D), jnp.float32),
            pltpu.VMEM((_CH, _D), jnp.float32),
            pltpu.VMEM_SHARED((_N_TOTAL, _D), jnp.float32),
            pltpu.SemaphoreType.DMA,
            pltpu.SemaphoreType.DMA,
            pltpu.SemaphoreType.DMA,
            pltpu.SemaphoreType.DMA,
            pltpu.SemaphoreType.DMA,
            pltpu.SemaphoreType.DMA,
            pltpu.SemaphoreType.DMA,
            pltpu.SemaphoreType.DMA,
        ],
    )
    def k(table_hbm, ed_hbm, dd_hbm, zeros_hbm, out_hbm,
          ed0, ed1, dd0, dd1, rows0, rows1, acc,
          gs0, gs1, ss0, ss1, xs0, xs1, ys0, ys1):
        c = lax.axis_index("c")
        s = lax.axis_index("s")
        wid = s * _NC + c
        row0 = pl.multiple_of(s * _RPS, 8)

        @pl.when(s < _NS - 1)
        def _():
            pltpu.sync_copy(zeros_hbm.at[pl.ds(row0, _RPS)],
                            acc.at[pl.ds(row0, _RPS)])

        @pl.when(s == _NS - 1)
        def _():
            pltpu.sync_copy(zeros_hbm.at[pl.ds(15 * _RPS, _RPS_LAST)],
                            acc.at[pl.ds(15 * _RPS, _RPS_LAST)])

        plsc.subcore_barrier()

        def scale(rows, ed):
            for g in range(ngrp):
                w16 = ed[1, pl.ds(g * 16, 16)].astype(jnp.float32) * (1.0 / (1 << 24))
                for j in range(16):
                    e = g * 16 + j
                    bw = _bcast_lane(w16, j)
                    for d16 in range(_D // 16):
                        sl = pl.ds(d16 * 16, 16)
                        rows[e, sl] = rows[e, sl] * bw

        def scatter(rows, dd, sem):
            pltpu.async_copy(rows, acc.at[dd.at[0]], sem, add=True)

        def drain_scatter(rows, dd, sem):
            pltpu.make_async_copy(rows, acc.at[dd.at[0]], sem).wait()

        def gather(ed, rows, sem):
            pltpu.async_copy(table_hbm.at[ed.at[0, pl.ds(0, _CH2)]],
                             rows.at[pl.ds(0, _CH2)], sem)
            pltpu.async_copy(table_hbm.at[ed.at[0, pl.ds(_CH2, _CH2)]],
                             rows.at[pl.ds(_CH2, _CH2)], sem)

        def wait_gather(ed, rows, sem):
            pltpu.make_async_copy(table_hbm.at[ed.at[0, pl.ds(0, _CH2)]],
                                  rows.at[pl.ds(0, _CH2)], sem).wait()
            pltpu.make_async_copy(table_hbm.at[ed.at[0, pl.ds(_CH2, _CH2)]],
                                  rows.at[pl.ds(_CH2, _CH2)], sem).wait()

        pltpu.sync_copy(ed_hbm.at[wid, 0], ed0)
        pltpu.sync_copy(ed_hbm.at[wid, 1], ed1)
        pltpu.sync_copy(dd_hbm.at[wid, 0], dd0)
        pltpu.sync_copy(dd_hbm.at[wid, 1], dd1)
        gather(ed0, rows0, gs0)

        def pair(p, carry):
            i0 = 2 * p
            i1 = i0 + 1

            @pl.when(p > 0)
            def _():
                drain_scatter(rows1, dd1, ss1)
                pltpu.async_copy(dd_hbm.at[wid, i1], dd1, ys1)
                pltpu.make_async_copy(ed_hbm.at[wid, i1], ed1, xs1).wait()

            gather(ed1, rows1, gs1)

            wait_gather(ed0, rows0, gs0)
            scale(rows0, ed0)

            @pl.when(p < _NP - 1)
            def _():
                pltpu.async_copy(ed_hbm.at[wid, i0 + 2], ed0, xs0)

            @pl.when(p > 0)
            def _():
                pltpu.make_async_copy(dd_hbm.at[wid, i0], dd0, ys0).wait()

            scatter(rows0, dd0, ss0)

            wait_gather(ed1, rows1, gs1)
            scale(rows1, ed1)
            drain_scatter(rows0, dd0, ss0)

            @pl.when(p < _NP - 1)
            def _():
                pltpu.async_copy(dd_hbm.at[wid, i0 + 2], dd0, ys0)
                pltpu.make_async_copy(ed_hbm.at[wid, i0 + 2], ed0, xs0).wait()
                gather(ed0, rows0, gs0)

            @pl.when(p > 0)
            def _():
                pltpu.make_async_copy(dd_hbm.at[wid, i1], dd1, ys1).wait()

            scatter(rows1, dd1, ss1)

            @pl.when(p < _NP - 1)
            def _():
                pltpu.async_copy(ed_hbm.at[wid, i1 + 2], ed1, xs1)

            return carry

        lax.fori_loop(0, _NP, pair, 0)
        drain_scatter(rows1, dd1, ss1)
        plsc.subcore_barrier()

        @pl.when(s < _NS - 1)
        def _():
            pltpu.sync_copy(acc.at[pl.ds(row0, _RPS)],
                            out_hbm.at[c, pl.ds(row0, _RPS)])

        @pl.when(s == _NS - 1)
        def _():
            pltpu.sync_copy(acc.at[pl.ds(15 * _RPS, _RPS_LAST)],
                            out_hbm.at[c, pl.ds(15 * _RPS, _RPS_LAST)])

    return k(table, edata, ddata, zeros)


def _combine(partials):
    br = 1200

    def body(p_ref, table_ref):
        table_ref[...] = p_ref[0] + p_ref[1]

    return pl.pallas_call(
        body,
        grid=(_N_TOTAL // br,),
        in_specs=[pl.BlockSpec((2, br, _D), lambda i: (0, i, 0))],
        out_specs=pl.BlockSpec((br, _D), lambda i: (i, 0)),
        out_shape=jax.ShapeDtypeStruct((_N_TOTAL, _D), jnp.float32),
    )(partials)


def _head(t0, t1, t2, p3, w, b2):
    br = 2000

    def body(t0_ref, t1_ref, t2_ref, p3_ref, w_ref, b_ref, o_ref):
        x = (t0_ref[...] + t1_ref[...] + t2_ref[...]
             + p3_ref[0] + p3_ref[1]) * (1.0 / (_L + 1))
        z = lax.dot_general(x, w_ref[...], (((1,), (1,)), ((), ())),
                            preferred_element_type=jnp.float32)
        z = z + b_ref[...]
        m = jnp.max(z, axis=1, keepdims=True)
        lse = jnp.log(jnp.sum(jnp.exp(z - m), axis=1, keepdims=True)) + m
        o_ref[...] = z - lse

    row_spec = pl.BlockSpec((br, _D), lambda i: (i, 0))
    return pl.pallas_call(
        body,
        grid=(_N_NODES // br,),
        in_specs=[
            row_spec,
            row_spec,
            row_spec,
            pl.BlockSpec((2, br, _D), lambda i: (0, i, 0)),
            pl.BlockSpec((_C, _D), lambda i: (0, 0)),
            pl.BlockSpec((1, _C), lambda i: (0, 0)),
        ],
        out_specs=pl.BlockSpec((br, _C), lambda i: (i, 0)),
        out_shape=jax.ShapeDtypeStruct((_N_NODES, _C), jnp.float32),
    )(t0, t1, t2, p3, w, b2)


def kernel(node_emb, hyperedge_emb, W, b, edge_weight, edge_index):
    table = jnp.concatenate([node_emb, hyperedge_emb], axis=0)
    dst = edge_index[0].astype(jnp.int32).reshape(_NW, _NCHUNK, _CH)
    src = edge_index[1].astype(jnp.int32).reshape(_NW, _NCHUNK, _CH)
    wbits = jnp.round(edge_weight.astype(jnp.float32) * (1 << 24)).astype(
        jnp.int32).reshape(_NW, _NCHUNK, _CH)
    edata = jnp.stack([src, wbits], axis=2)
    ddata = dst.reshape(_NW, _NCHUNK, 1, _CH)
    zeros = jnp.zeros((_N_TOTAL, _D), jnp.float32)

    tables = [table]
    for _ in range(_L - 1):
        partials = _sc_layer(tables[-1], edata, ddata, zeros)
        tables.append(_combine(partials))
    p3 = _sc_layer(tables[-1], edata, ddata, zeros)

    n = _N_NODES
    return _head(tables[0][:n], tables[1][:n], tables[2][:n], p3[:, :n],
                 W, jnp.reshape(b, (1, _C)))
